# Initial kernel scaffold; baseline (speedup 1.0000x reference)
import jax
import jax.numpy as jnp
from jax.experimental import pallas as pl


def kernel(xyz_q, lat_rep, xyz, points, w_qs, w_ks, w_vs, w_kg, w_vg, wd1, bd1, wd2, bd2, wg1, bg1, wg2, bg2, wp, bp, wc, bc, wb0, bb0, wb1, bb1):
    raise NotImplementedError("write your pallas kernel here")


# speedup vs baseline: 15.9499x; 15.9499x over previous
import functools

import numpy as np

import jax
import jax.numpy as jnp
from jax import lax
from jax.experimental import pallas as pl
from jax.experimental.pallas import tpu as pltpu
from jax.experimental.pallas import tpu_sc as plsc

F32 = jnp.float32
QB = 256
TW = 384
NNB = 16


def _table_body(points_ref, wk_ref, wv_ref, xyz_ref, table_ref):
    pts = points_ref[0]
    table_ref[0, :, 0:128] = jnp.dot(pts, wk_ref[...], preferred_element_type=F32)
    table_ref[0, :, 128:256] = jnp.dot(pts, wv_ref[...], preferred_element_type=F32)
    xyz = xyz_ref[0]
    pad = jnp.zeros((xyz.shape[0], TW - 256 - 3), F32)
    table_ref[0, :, 256:TW] = jnp.concatenate([xyz, pad], axis=1)


def _build_table(points, w_ks, w_vs, xyz):
    b, n, _ = points.shape
    return pl.pallas_call(
        _table_body,
        grid=(b,),
        in_specs=[
            pl.BlockSpec((1, n, 128), lambda i: (i, 0, 0)),
            pl.BlockSpec((128, 128), lambda i: (0, 0)),
            pl.BlockSpec((128, 128), lambda i: (0, 0)),
            pl.BlockSpec((1, n, 3), lambda i: (i, 0, 0)),
        ],
        out_specs=pl.BlockSpec((1, n, TW), lambda i: (i, 0, 0)),
        out_shape=jax.ShapeDtypeStruct((b, n, TW), F32),
    )(points, w_ks, w_vs, xyz)


def _knn_body(n, xq_ref, xt_ref, idx_ref):
    bi = pl.program_id(0)
    xt = xt_ref[0]
    x2 = xt[0:1] * xt[0:1] + xt[1:2] * xt[1:2] + xt[2:3] * xt[2:3]
    q = xq_ref[0]
    s2 = q[:, 0:1] * q[:, 0:1] + q[:, 1:2] * q[:, 1:2] + q[:, 2:3] * q[:, 2:3]
    dot = jnp.dot(q.astype(jnp.bfloat16), xt.astype(jnp.bfloat16),
                  preferred_element_type=F32)
    d = (s2 + x2) - 2.0 * dot
    iota = lax.broadcasted_iota(jnp.int32, d.shape, 1)
    cols = []
    for _ in range(NNB):
        m = jnp.min(d, axis=1, keepdims=True)
        cand = jnp.where(d == m, iota, jnp.int32(n))
        idx = jnp.min(cand, axis=1, keepdims=True)
        cols.append(idx)
        d = jnp.where(iota == idx, F32(3.0e38), d)
    idx_ref[0] = jnp.concatenate(cols, axis=1) + bi * n


def _knn(xyz_q, xyz_t):
    b, nq, _ = xyz_q.shape
    n = xyz_t.shape[2]
    return pl.pallas_call(
        functools.partial(_knn_body, n),
        grid=(b, nq // QB),
        in_specs=[
            pl.BlockSpec((1, QB, 3), lambda i, j: (i, j, 0)),
            pl.BlockSpec((1, 3, n), lambda i, j: (i, 0, 0)),
        ],
        out_specs=pl.BlockSpec((1, QB, NNB), lambda i, j: (i, j, 0)),
        out_shape=jax.ShapeDtypeStruct((b, nq, NNB), jnp.int32),
    )(xyz_q, xyz_t)


@functools.lru_cache(maxsize=None)
def _make_sc_gather(tot_rows, tot_idx):
    nw = 32
    per_w = tot_idx // nw
    ch = 128
    nch = per_w // ch

    @functools.partial(
        pl.kernel,
        out_type=jax.ShapeDtypeStruct((tot_idx, TW), F32),
        mesh=plsc.VectorSubcoreMesh(core_axis_name="c", subcore_axis_name="s"),
        scratch_types=[
            pltpu.VMEM((per_w,), jnp.int32),
            pltpu.VMEM((ch, TW), F32),
            pltpu.VMEM((ch, TW), F32),
            pltpu.SemaphoreType.DMA,
            pltpu.SemaphoreType.DMA,
        ],
    )
    def gather(table_hbm, idx_hbm, out_hbm, idx_v, buf0, buf1, sem0, sem1):
        wid = lax.axis_index("s") * 2 + lax.axis_index("c")
        base = wid * per_w
        pltpu.sync_copy(idx_hbm.at[pl.ds(base, per_w)], idx_v)
        bufs, sems, cps = (buf0, buf1), (sem0, sem1), [None, None]
        for c in range(nch):
            cps[c % 2] = pltpu.async_copy(
                table_hbm.at[idx_v.at[pl.ds(c * ch, ch)]], bufs[c % 2], sems[c % 2])
            if c > 0:
                cps[(c - 1) % 2].wait()
                pltpu.sync_copy(bufs[(c - 1) % 2],
                                out_hbm.at[pl.ds(base + (c - 1) * ch, ch)])
        cps[(nch - 1) % 2].wait()
        pltpu.sync_copy(bufs[(nch - 1) % 2],
                        out_hbm.at[pl.ds(base + (nch - 1) * ch, ch)])

    return gather


def _sc_gather(table_flat, idx_flat):
    return _make_sc_gather(table_flat.shape[0], idx_flat.shape[0])(
        table_flat, idx_flat)


def _attn_body(g_ref, xq_ref, lr_ref, wqs_ref, wkg_ref, wvg_ref,
               wd1_ref, bd1_ref, wd2_ref, bd2_ref,
               wg1_ref, bg1_ref, wg2_ref, bg2_ref,
               fm_ref, wps_ref, wpc_ref, bp_ref,
               wc_ref, bc_ref, wb0_ref, bb0_ref, wb1_ref, bb1_ref, out_ref):
    relu = jax.nn.relu
    mm = lambda a, w: jnp.dot(a, w, preferred_element_type=F32)
    g = g_ref[...]
    k_nb = g[:, 0:128]
    v_nb = g[:, 128:256]
    xk = g[:, 256:259]
    lr = lr_ref[0]
    qa = mm(lr, wqs_ref[...])
    kg = mm(lr, wkg_ref[...])
    vg = mm(lr, wvg_ref[...])
    xq = xq_ref[0]
    xqr = jnp.broadcast_to(xq[:, None, :], (QB, NNB, 3)).reshape(QB * NNB, 3)
    d3 = xqr - xk
    wd1 = wd1_ref[...]
    h1 = relu(d3[:, 0:1] * wd1[0:1] + d3[:, 1:2] * wd1[1:2]
              + d3[:, 2:3] * wd1[2:3] + bd1_ref[...])
    pos = mm(h1, wd2_ref[...]) + bd2_ref[...]
    pre = qa - k_nb + pos
    a_nb = mm(relu(mm(pre, wg1_ref[...]) + bg1_ref[...]), wg2_ref[...]) + bg2_ref[...]
    a_g = mm(relu(mm(qa - kg, wg1_ref[...]) + bg1_ref[...]), wg2_ref[...]) + bg2_ref[...]
    a3 = a_nb.reshape(QB, NNB, 128)
    mx = jnp.maximum(jnp.max(a3, axis=1), a_g)
    e = jnp.exp(a3 - mx[:, None, :])
    eg = jnp.exp(a_g - mx)
    s = jnp.sum(e, axis=1) + eg
    vp3 = (v_nb + pos).reshape(QB, NNB, 128)
    res = (jnp.sum(e * vp3, axis=1) + eg * vg) / s
    fm = fm_ref[...]
    ang = xq[:, 0:1] * fm[0:1] + xq[:, 1:2] * fm[1:2] + xq[:, 2:3] * fm[2:3]
    net = mm(jnp.sin(ang), wps_ref[...]) + mm(jnp.cos(ang), wpc_ref[...]) + bp_ref[...]
    for i in range(wc_ref.shape[0]):
        net = net + mm(res, wc_ref[i]) + bc_ref[i:i + 1]
        h = relu(net)
        h = mm(h, wb0_ref[i]) + bb0_ref[i:i + 1]
        net = net + mm(relu(h), wb1_ref[i]) + bb1_ref[i:i + 1]
    out_ref[0] = net


def _attn_decoder(gathered, xyz_q, lat_rep, w_qs, w_kg, w_vg,
                  wd1, bd1, wd2, bd2, wg1, bg1, wg2, bg2,
                  fmat, wp_sin, wp_cos, bp, wc, bc, wb0, bb0, wb1, bb1):
    b, nq, _ = xyz_q.shape
    nqb = nq // QB
    full = lambda shape: pl.BlockSpec(shape, lambda i, j: tuple(0 for _ in shape))
    return pl.pallas_call(
        _attn_body,
        grid=(b, nqb),
        in_specs=[
            pl.BlockSpec((QB * NNB, TW), lambda i, j: (i * nqb + j, 0)),
            pl.BlockSpec((1, QB, 3), lambda i, j: (i, j, 0)),
            pl.BlockSpec((1, 1, 128), lambda i, j: (i, 0, 0)),
            full((128, 128)), full((128, 128)), full((128, 128)),
            full((3, 128)), full((1, 128)), full((128, 128)), full((1, 128)),
            full((128, 128)), full((1, 128)), full((128, 128)), full((1, 128)),
            full((3, 30)), full((30, 128)), full((30, 128)), full((1, 128)),
            full((5, 128, 128)), full((5, 128)),
            full((5, 128, 128)), full((5, 128)),
            full((5, 128, 128)), full((5, 128)),
        ],
        out_specs=pl.BlockSpec((1, QB, 128), lambda i, j: (i, j, 0)),
        out_shape=jax.ShapeDtypeStruct((b, nq, 128), F32),
    )(gathered, xyz_q, lat_rep, w_qs, w_kg, w_vg,
      wd1, bd1, wd2, bd2, wg1, bg1, wg2, bg2,
      fmat, wp_sin, wp_cos, bp, wc, bc, wb0, bb0, wb1, bb1)


def _pe_constants():
    freqs = (2.0 ** np.arange(10, dtype=np.float32)) * np.pi
    fm = np.zeros((3, 30), np.float32)
    for f in range(10):
        for c in range(3):
            fm[c, f * 3 + c] = freqs[f]
    perm_sin = np.array([f * 6 + c for f in range(10) for c in range(3)], np.int32)
    perm_cos = perm_sin + 3
    return jnp.asarray(fm), perm_sin, perm_cos


def kernel(xyz_q, lat_rep, xyz, points, w_qs, w_ks, w_vs, w_kg, w_vg,
           wd1, bd1, wd2, bd2, wg1, bg1, wg2, bg2, wp, bp,
           wc, bc, wb0, bb0, wb1, bb1):
    b, nq, _ = xyz_q.shape
    n = xyz.shape[1]
    table = _build_table(points, w_ks, w_vs, xyz)
    xyz_t = jnp.swapaxes(xyz, 1, 2)
    idx = _knn(xyz_q, xyz_t)
    gathered = _sc_gather(table.reshape(b * n, TW),
                          idx.reshape(b * nq * NNB))
    fmat, perm_sin, perm_cos = _pe_constants()
    out = _attn_decoder(
        gathered, xyz_q, lat_rep.reshape(b, 1, 128), w_qs, w_kg, w_vg,
        wd1, bd1.reshape(1, 128), wd2, bd2.reshape(1, 128),
        wg1, bg1.reshape(1, 128), wg2, bg2.reshape(1, 128),
        fmat, wp[perm_sin], wp[perm_cos], bp.reshape(1, 128),
        wc, bc, wb0, bb0, wb1, bb1)
    return out


# speedup vs baseline: 16.3971x; 1.0280x over previous
import functools

import numpy as np

import jax
import jax.numpy as jnp
from jax import lax
from jax.experimental import pallas as pl
from jax.experimental.pallas import tpu as pltpu
from jax.experimental.pallas import tpu_sc as plsc

F32 = jnp.float32
QB = 256
TW = 384
NNB = 16


def _table_body(points_ref, wk_ref, wv_ref, xyz_ref, table_ref):
    pts = points_ref[0]
    table_ref[0, :, 0:128] = jnp.dot(pts, wk_ref[...], preferred_element_type=F32)
    table_ref[0, :, 128:256] = jnp.dot(pts, wv_ref[...], preferred_element_type=F32)
    xyz = xyz_ref[0]
    pad = jnp.zeros((xyz.shape[0], TW - 256 - 3), F32)
    table_ref[0, :, 256:TW] = jnp.concatenate([xyz, pad], axis=1)


def _build_table(points, w_ks, w_vs, xyz):
    b, n, _ = points.shape
    return pl.pallas_call(
        _table_body,
        grid=(b,),
        in_specs=[
            pl.BlockSpec((1, n, 128), lambda i: (i, 0, 0)),
            pl.BlockSpec((128, 128), lambda i: (0, 0)),
            pl.BlockSpec((128, 128), lambda i: (0, 0)),
            pl.BlockSpec((1, n, 3), lambda i: (i, 0, 0)),
        ],
        out_specs=pl.BlockSpec((1, n, TW), lambda i: (i, 0, 0)),
        out_shape=jax.ShapeDtypeStruct((b, n, TW), F32),
    )(points, w_ks, w_vs, xyz)


def _knn_body(n, xq_ref, xt_ref, idx_ref):
    bi = pl.program_id(0)
    xt = xt_ref[0]
    x2 = xt[0:1] * xt[0:1] + xt[1:2] * xt[1:2] + xt[2:3] * xt[2:3]
    q = xq_ref[0]
    s2 = q[:, 0:1] * q[:, 0:1] + q[:, 1:2] * q[:, 1:2] + q[:, 2:3] * q[:, 2:3]
    dot = jnp.dot(q.astype(jnp.bfloat16), xt.astype(jnp.bfloat16),
                  preferred_element_type=F32)
    d = (s2 + x2) - 2.0 * dot
    iota = lax.broadcasted_iota(jnp.int32, d.shape, 1)
    cols = []
    for _ in range(NNB):
        m = jnp.min(d, axis=1, keepdims=True)
        cand = jnp.where(d == m, iota, jnp.int32(n))
        idx = jnp.min(cand, axis=1, keepdims=True)
        cols.append(idx)
        d = jnp.where(iota == idx, F32(3.0e38), d)
    idx_ref[0] = jnp.concatenate(cols, axis=1) + bi * n


def _knn(xyz_q, xyz_t):
    b, nq, _ = xyz_q.shape
    n = xyz_t.shape[2]
    return pl.pallas_call(
        functools.partial(_knn_body, n),
        grid=(b, nq // QB),
        in_specs=[
            pl.BlockSpec((1, QB, 3), lambda i, j: (i, j, 0)),
            pl.BlockSpec((1, 3, n), lambda i, j: (i, 0, 0)),
        ],
        out_specs=pl.BlockSpec((1, QB, NNB), lambda i, j: (i, j, 0)),
        out_shape=jax.ShapeDtypeStruct((b, nq, NNB), jnp.int32),
    )(xyz_q, xyz_t)


@functools.lru_cache(maxsize=None)
def _make_sc_gather(tot_rows, tot_idx):
    nw = 32
    per_w = tot_idx // nw
    ch = 128
    nch = per_w // ch

    @functools.partial(
        pl.kernel,
        out_type=jax.ShapeDtypeStruct((tot_idx, TW), F32),
        mesh=plsc.VectorSubcoreMesh(core_axis_name="c", subcore_axis_name="s"),
        scratch_types=[
            pltpu.VMEM((per_w,), jnp.int32),
            pltpu.VMEM((ch, TW), F32),
            pltpu.VMEM((ch, TW), F32),
            pltpu.SemaphoreType.DMA,
            pltpu.SemaphoreType.DMA,
        ],
    )
    def gather(table_hbm, idx_hbm, out_hbm, idx_v, buf0, buf1, sem0, sem1):
        wid = lax.axis_index("s") * 2 + lax.axis_index("c")
        base = wid * per_w
        pltpu.sync_copy(idx_hbm.at[pl.ds(base, per_w)], idx_v)
        bufs, sems, cps = (buf0, buf1), (sem0, sem1), [None, None]
        for c in range(nch):
            cps[c % 2] = pltpu.async_copy(
                table_hbm.at[idx_v.at[pl.ds(c * ch, ch)]], bufs[c % 2], sems[c % 2])
            if c > 0:
                cps[(c - 1) % 2].wait()
                pltpu.sync_copy(bufs[(c - 1) % 2],
                                out_hbm.at[pl.ds(base + (c - 1) * ch, ch)])
        cps[(nch - 1) % 2].wait()
        pltpu.sync_copy(bufs[(nch - 1) % 2],
                        out_hbm.at[pl.ds(base + (nch - 1) * ch, ch)])

    return gather


def _sc_gather(table_flat, idx_flat):
    return _make_sc_gather(table_flat.shape[0], idx_flat.shape[0])(
        table_flat, idx_flat)


def _attn_body(g_ref, xq_ref, lr_ref, wqs_ref, wkg_ref, wvg_ref,
               wd1_ref, bd1_ref, wd2_ref, bd2_ref,
               wg1_ref, bg1_ref, wg2_ref, bg2_ref,
               fm_ref, wps_ref, wpc_ref, bp_ref,
               wc_ref, bc_ref, wb0_ref, bb0_ref, wb1_ref, bb1_ref, out_ref):
    relu = jax.nn.relu
    mm = lambda a, w: jnp.dot(a, w, preferred_element_type=F32)
    g = g_ref[...]
    k_nb = g[:, 0:128]
    v_nb = g[:, 128:256]
    xk = g[:, 256:259]
    lr = lr_ref[0]
    qa = mm(lr, wqs_ref[...])
    kg = mm(lr, wkg_ref[...])
    vg = mm(lr, wvg_ref[...])
    xq = xq_ref[0]
    xqr = jnp.broadcast_to(xq[:, None, :], (QB, NNB, 3)).reshape(QB * NNB, 3)
    d3 = xqr - xk
    wd1 = wd1_ref[...]
    h1 = relu(d3[:, 0:1] * wd1[0:1] + d3[:, 1:2] * wd1[1:2]
              + d3[:, 2:3] * wd1[2:3] + bd1_ref[...])
    pos = mm(h1, wd2_ref[...]) + bd2_ref[...]
    pre = qa - k_nb + pos
    a_nb = mm(relu(mm(pre, wg1_ref[...]) + bg1_ref[...]), wg2_ref[...]) + bg2_ref[...]
    a_g = mm(relu(mm(qa - kg, wg1_ref[...]) + bg1_ref[...]), wg2_ref[...]) + bg2_ref[...]
    a3 = a_nb.reshape(QB, NNB, 128)
    mx = jnp.maximum(jnp.max(a3, axis=1), a_g)
    e = jnp.exp(a3 - mx[:, None, :])
    eg = jnp.exp(a_g - mx)
    s = jnp.sum(e, axis=1) + eg
    vp3 = (v_nb + pos).reshape(QB, NNB, 128)
    res = (jnp.sum(e * vp3, axis=1) + eg * vg) / s
    fm = fm_ref[...]
    ang = xq[:, 0:1] * fm[0:1] + xq[:, 1:2] * fm[1:2] + xq[:, 2:3] * fm[2:3]
    net = mm(jnp.sin(ang), wps_ref[...]) + mm(jnp.cos(ang), wpc_ref[...]) + bp_ref[...]
    for i in range(wc_ref.shape[0]):
        net = net + mm(res, wc_ref[i]) + bc_ref[i:i + 1]
        h = relu(net)
        h = mm(h, wb0_ref[i]) + bb0_ref[i:i + 1]
        net = net + mm(relu(h), wb1_ref[i]) + bb1_ref[i:i + 1]
    out_ref[0] = net


def _attn_decoder(gathered, xyz_q, lat_rep, w_qs, w_kg, w_vg,
                  wd1, bd1, wd2, bd2, wg1, bg1, wg2, bg2,
                  fmat, wp_sin, wp_cos, bp, wc, bc, wb0, bb0, wb1, bb1):
    b, nq, _ = xyz_q.shape
    nqb = nq // QB
    full = lambda shape: pl.BlockSpec(shape, lambda i, j: tuple(0 for _ in shape))
    return pl.pallas_call(
        _attn_body,
        grid=(b, nqb),
        in_specs=[
            pl.BlockSpec((QB * NNB, TW), lambda i, j: (i * nqb + j, 0)),
            pl.BlockSpec((1, QB, 3), lambda i, j: (i, j, 0)),
            pl.BlockSpec((1, 1, 128), lambda i, j: (i, 0, 0)),
            full((128, 128)), full((128, 128)), full((128, 128)),
            full((3, 128)), full((1, 128)), full((128, 128)), full((1, 128)),
            full((128, 128)), full((1, 128)), full((128, 128)), full((1, 128)),
            full((3, 30)), full((30, 128)), full((30, 128)), full((1, 128)),
            full((5, 128, 128)), full((5, 128)),
            full((5, 128, 128)), full((5, 128)),
            full((5, 128, 128)), full((5, 128)),
        ],
        out_specs=pl.BlockSpec((1, QB, 128), lambda i, j: (i, j, 0)),
        out_shape=jax.ShapeDtypeStruct((b, nq, 128), F32),
    )(gathered, xyz_q, lat_rep, w_qs, w_kg, w_vg,
      wd1, bd1, wd2, bd2, wg1, bg1, wg2, bg2,
      fmat, wp_sin, wp_cos, bp, wc, bc, wb0, bb0, wb1, bb1)


def _pe_constants():
    freqs = (2.0 ** np.arange(10, dtype=np.float32)) * np.pi
    fm = np.zeros((3, 30), np.float32)
    for f in range(10):
        for c in range(3):
            fm[c, f * 3 + c] = freqs[f]
    perm_sin = np.array([f * 6 + c for f in range(10) for c in range(3)], np.int32)
    perm_cos = perm_sin + 3
    return jnp.asarray(fm), perm_sin, perm_cos


def kernel(xyz_q, lat_rep, xyz, points, w_qs, w_ks, w_vs, w_kg, w_vg,
           wd1, bd1, wd2, bd2, wg1, bg1, wg2, bg2, wp, bp,
           wc, bc, wb0, bb0, wb1, bb1):
    b, nq, _ = xyz_q.shape
    n = xyz.shape[1]
    table = _build_table(points, w_ks, w_vs, xyz)
    table_flat = table.reshape(b * n, TW)
    xyz_t = jnp.swapaxes(xyz, 1, 2)
    fmat, perm_sin, perm_cos = _pe_constants()

    halves = (xyz_q[:, : nq // 2], xyz_q[:, nq // 2:])
    idxs = [_knn(h, xyz_t) for h in halves]
    gathers = [_sc_gather(table_flat, ix.reshape(-1)) for ix in idxs]
    outs = [
        _attn_decoder(
            g, h, lat_rep.reshape(b, 1, 128), w_qs, w_kg, w_vg,
            wd1, bd1.reshape(1, 128), wd2, bd2.reshape(1, 128),
            wg1, bg1.reshape(1, 128), wg2, bg2.reshape(1, 128),
            fmat, wp[perm_sin], wp[perm_cos], bp.reshape(1, 128),
            wc, bc, wb0, bb0, wb1, bb1)
        for g, h in zip(gathers, halves)
    ]
    return jnp.concatenate(outs, axis=1)


# speedup vs baseline: 20.3816x; 1.2430x over previous
import functools

import numpy as np

import jax
import jax.numpy as jnp
from jax import lax
from jax.experimental import pallas as pl
from jax.experimental.pallas import tpu as pltpu
from jax.experimental.pallas import tpu_sc as plsc

F32 = jnp.float32
QB = 256
TW = 384
NNB = 16


def _table_body(points_ref, wk_ref, wv_ref, xyz_ref, table_ref):
    pts = points_ref[0]
    table_ref[0, :, 0:128] = jnp.dot(pts, wk_ref[...], preferred_element_type=F32)
    table_ref[0, :, 128:256] = jnp.dot(pts, wv_ref[...], preferred_element_type=F32)
    xyz = xyz_ref[0]
    pad = jnp.zeros((xyz.shape[0], TW - 256 - 3), F32)
    table_ref[0, :, 256:TW] = jnp.concatenate([xyz, pad], axis=1)


def _build_table(points, w_ks, w_vs, xyz):
    b, n, _ = points.shape
    return pl.pallas_call(
        _table_body,
        grid=(b,),
        in_specs=[
            pl.BlockSpec((1, n, 128), lambda i: (i, 0, 0)),
            pl.BlockSpec((128, 128), lambda i: (0, 0)),
            pl.BlockSpec((128, 128), lambda i: (0, 0)),
            pl.BlockSpec((1, n, 3), lambda i: (i, 0, 0)),
        ],
        out_specs=pl.BlockSpec((1, n, TW), lambda i: (i, 0, 0)),
        out_shape=jax.ShapeDtypeStruct((b, n, TW), F32),
    )(points, w_ks, w_vs, xyz)


def _knn_body(n, xq_ref, xt_ref, idx_ref):
    bi = pl.program_id(0)
    xt = xt_ref[0]
    x2 = xt[0:1] * xt[0:1] + xt[1:2] * xt[1:2] + xt[2:3] * xt[2:3]
    q = xq_ref[0]
    s2 = q[:, 0:1] * q[:, 0:1] + q[:, 1:2] * q[:, 1:2] + q[:, 2:3] * q[:, 2:3]
    dot = jnp.dot(q.astype(jnp.bfloat16), xt.astype(jnp.bfloat16),
                  preferred_element_type=F32)
    d = (s2 + x2) - 2.0 * dot
    iota = lax.broadcasted_iota(jnp.int32, d.shape, 1).astype(F32)
    nf = F32(n)
    cols = []
    for _ in range(NNB):
        m = jnp.min(d, axis=1, keepdims=True)
        hit = d == m
        idx = jnp.min(jnp.where(hit, iota, nf), axis=1, keepdims=True)
        cols.append(idx)
        d = jnp.where(hit, F32(3.0e38), d)
    idx_ref[0] = jnp.concatenate(cols, axis=1).astype(jnp.int32) + bi * n


QBK = 128


def _knn(xyz_q, xyz_t):
    b, nq, _ = xyz_q.shape
    n = xyz_t.shape[2]
    return pl.pallas_call(
        functools.partial(_knn_body, n),
        grid=(b, nq // QBK),
        in_specs=[
            pl.BlockSpec((1, QBK, 3), lambda i, j: (i, j, 0)),
            pl.BlockSpec((1, 3, n), lambda i, j: (i, 0, 0)),
        ],
        out_specs=pl.BlockSpec((1, QBK, NNB), lambda i, j: (i, j, 0)),
        out_shape=jax.ShapeDtypeStruct((b, nq, NNB), jnp.int32),
    )(xyz_q, xyz_t)


@functools.lru_cache(maxsize=None)
def _make_sc_gather(tot_rows, tot_idx):
    nw = 32
    per_w = tot_idx // nw
    ch = 128
    nch = per_w // ch

    @functools.partial(
        pl.kernel,
        out_type=jax.ShapeDtypeStruct((tot_idx, TW), F32),
        mesh=plsc.VectorSubcoreMesh(core_axis_name="c", subcore_axis_name="s"),
        scratch_types=[
            pltpu.VMEM((per_w,), jnp.int32),
            pltpu.VMEM((ch, TW), F32),
            pltpu.VMEM((ch, TW), F32),
            pltpu.SemaphoreType.DMA,
            pltpu.SemaphoreType.DMA,
        ],
    )
    def gather(table_hbm, idx_hbm, out_hbm, idx_v, buf0, buf1, sem0, sem1):
        wid = lax.axis_index("s") * 2 + lax.axis_index("c")
        base = wid * per_w
        pltpu.sync_copy(idx_hbm.at[pl.ds(base, per_w)], idx_v)
        bufs, sems, cps = (buf0, buf1), (sem0, sem1), [None, None]
        for c in range(nch):
            cps[c % 2] = pltpu.async_copy(
                table_hbm.at[idx_v.at[pl.ds(c * ch, ch)]], bufs[c % 2], sems[c % 2])
            if c > 0:
                cps[(c - 1) % 2].wait()
                pltpu.sync_copy(bufs[(c - 1) % 2],
                                out_hbm.at[pl.ds(base + (c - 1) * ch, ch)])
        cps[(nch - 1) % 2].wait()
        pltpu.sync_copy(bufs[(nch - 1) % 2],
                        out_hbm.at[pl.ds(base + (nch - 1) * ch, ch)])

    return gather


def _sc_gather(table_flat, idx_flat):
    return _make_sc_gather(table_flat.shape[0], idx_flat.shape[0])(
        table_flat, idx_flat)


def _attn_body(g_ref, xq_ref, lr_ref, wqs_ref, wkg_ref, wvg_ref,
               wd1_ref, bd1_ref, wd2_ref, bd2_ref,
               wg1_ref, bg1_ref, wg2_ref, bg2_ref,
               fm_ref, wps_ref, wpc_ref, bp_ref,
               wc_ref, bc_ref, wb0_ref, bb0_ref, wb1_ref, bb1_ref, out_ref):
    relu = jax.nn.relu
    mm = lambda a, w: jnp.dot(a, w, preferred_element_type=F32)
    g = g_ref[...]
    k_nb = g[:, 0:128]
    v_nb = g[:, 128:256]
    xk = g[:, 256:259]
    lr = lr_ref[0]
    qa = mm(lr, wqs_ref[...])
    kg = mm(lr, wkg_ref[...])
    vg = mm(lr, wvg_ref[...])
    xq = xq_ref[0]
    xqr = jnp.broadcast_to(xq[:, None, :], (QB, NNB, 3)).reshape(QB * NNB, 3)
    d3 = xqr - xk
    wd1 = wd1_ref[...]
    h1 = relu(d3[:, 0:1] * wd1[0:1] + d3[:, 1:2] * wd1[1:2]
              + d3[:, 2:3] * wd1[2:3] + bd1_ref[...])
    pos = mm(h1, wd2_ref[...]) + bd2_ref[...]
    pre = qa - k_nb + pos
    a_nb = mm(relu(mm(pre, wg1_ref[...]) + bg1_ref[...]), wg2_ref[...]) + bg2_ref[...]
    a_g = mm(relu(mm(qa - kg, wg1_ref[...]) + bg1_ref[...]), wg2_ref[...]) + bg2_ref[...]
    a3 = a_nb.reshape(QB, NNB, 128)
    mx = jnp.maximum(jnp.max(a3, axis=1), a_g)
    e = jnp.exp(a3 - mx[:, None, :])
    eg = jnp.exp(a_g - mx)
    s = jnp.sum(e, axis=1) + eg
    vp3 = (v_nb + pos).reshape(QB, NNB, 128)
    res = (jnp.sum(e * vp3, axis=1) + eg * vg) / s
    fm = fm_ref[...]
    ang = xq[:, 0:1] * fm[0:1] + xq[:, 1:2] * fm[1:2] + xq[:, 2:3] * fm[2:3]
    net = mm(jnp.sin(ang), wps_ref[...]) + mm(jnp.cos(ang), wpc_ref[...]) + bp_ref[...]
    for i in range(wc_ref.shape[0]):
        net = net + mm(res, wc_ref[i]) + bc_ref[i:i + 1]
        h = relu(net)
        h = mm(h, wb0_ref[i]) + bb0_ref[i:i + 1]
        net = net + mm(relu(h), wb1_ref[i]) + bb1_ref[i:i + 1]
    out_ref[0] = net


def _attn_decoder(gathered, xyz_q, lat_rep, w_qs, w_kg, w_vg,
                  wd1, bd1, wd2, bd2, wg1, bg1, wg2, bg2,
                  fmat, wp_sin, wp_cos, bp, wc, bc, wb0, bb0, wb1, bb1):
    b, nq, _ = xyz_q.shape
    nqb = nq // QB
    full = lambda shape: pl.BlockSpec(shape, lambda i, j: tuple(0 for _ in shape))
    return pl.pallas_call(
        _attn_body,
        grid=(b, nqb),
        in_specs=[
            pl.BlockSpec((QB * NNB, TW), lambda i, j: (i * nqb + j, 0)),
            pl.BlockSpec((1, QB, 3), lambda i, j: (i, j, 0)),
            pl.BlockSpec((1, 1, 128), lambda i, j: (i, 0, 0)),
            full((128, 128)), full((128, 128)), full((128, 128)),
            full((3, 128)), full((1, 128)), full((128, 128)), full((1, 128)),
            full((128, 128)), full((1, 128)), full((128, 128)), full((1, 128)),
            full((3, 30)), full((30, 128)), full((30, 128)), full((1, 128)),
            full((5, 128, 128)), full((5, 128)),
            full((5, 128, 128)), full((5, 128)),
            full((5, 128, 128)), full((5, 128)),
        ],
        out_specs=pl.BlockSpec((1, QB, 128), lambda i, j: (i, j, 0)),
        out_shape=jax.ShapeDtypeStruct((b, nq, 128), F32),
    )(gathered, xyz_q, lat_rep, w_qs, w_kg, w_vg,
      wd1, bd1, wd2, bd2, wg1, bg1, wg2, bg2,
      fmat, wp_sin, wp_cos, bp, wc, bc, wb0, bb0, wb1, bb1)


def _pe_constants():
    freqs = (2.0 ** np.arange(10, dtype=np.float32)) * np.pi
    fm = np.zeros((3, 30), np.float32)
    for f in range(10):
        for c in range(3):
            fm[c, f * 3 + c] = freqs[f]
    perm_sin = np.array([f * 6 + c for f in range(10) for c in range(3)], np.int32)
    perm_cos = perm_sin + 3
    return jnp.asarray(fm), perm_sin, perm_cos


def kernel(xyz_q, lat_rep, xyz, points, w_qs, w_ks, w_vs, w_kg, w_vg,
           wd1, bd1, wd2, bd2, wg1, bg1, wg2, bg2, wp, bp,
           wc, bc, wb0, bb0, wb1, bb1):
    b, nq, _ = xyz_q.shape
    n = xyz.shape[1]
    table = _build_table(points, w_ks, w_vs, xyz)
    table_flat = table.reshape(b * n, TW)
    xyz_t = jnp.swapaxes(xyz, 1, 2)
    fmat, perm_sin, perm_cos = _pe_constants()

    halves = (xyz_q[:, : nq // 2], xyz_q[:, nq // 2:])
    idxs = [_knn(h, xyz_t) for h in halves]
    gathers = [_sc_gather(table_flat, ix.reshape(-1)) for ix in idxs]
    outs = [
        _attn_decoder(
            g, h, lat_rep.reshape(b, 1, 128), w_qs, w_kg, w_vg,
            wd1, bd1.reshape(1, 128), wd2, bd2.reshape(1, 128),
            wg1, bg1.reshape(1, 128), wg2, bg2.reshape(1, 128),
            fmat, wp[perm_sin], wp[perm_cos], bp.reshape(1, 128),
            wc, bc, wb0, bb0, wb1, bb1)
        for g, h in zip(gathers, halves)
    ]
    return jnp.concatenate(outs, axis=1)
